# baseline (speedup 1.0000x reference)
import jax
import jax.numpy as jnp
from jax.experimental import pallas as pl
from jax.experimental.pallas import tpu as pltpu

_N, _D, _H, _C = 10000, 128, 128, 2
_TILE = 2048


def _mask_kernel(x_ref, w_ref, v_ref, gut_ref, o_ref):
    w1 = w_ref[:, 0:_H]
    w2 = w_ref[:, _H:2 * _H]
    wc = w_ref[:, 2 * _H:2 * _H + _C]
    b1 = v_ref[0:1, :]
    b2 = v_ref[1:2, :]
    bc = v_ref[2:3, 0:_C]
    a = v_ref[2, _C]
    xt = x_ref[...].T
    h = jnp.dot(w1.T, xt, preferred_element_type=jnp.float32)
    h = h + b1.T
    h = jnp.where(h >= 0.0, h, a * h)
    h = jnp.dot(w2.T, h, preferred_element_type=jnp.float32)
    h = h + b2.T
    s = jnp.dot(wc.T, h, preferred_element_type=jnp.float32)
    s = s + bc.T
    g = -jnp.log(-jnp.log(gut_ref[...]))
    s = s + g
    m = jnp.max(s, axis=0, keepdims=True)
    e = jnp.exp(s - m)
    y = e / jnp.sum(e, axis=0, keepdims=True)
    hard1 = (y[1:2, :] > y[0:1, :]).astype(jnp.float32)
    o_ref[...] = jnp.concatenate([1.0 - hard1, hard1], axis=0)


def kernel(x, edge_index, W1, b1, prelu_a, W2, b2, Wc, bc, gumbel_u):
    del edge_index
    gut = gumbel_u.T
    wpack = jnp.concatenate([W1, W2, Wc], axis=1)
    vrow = jnp.concatenate([bc, prelu_a,
                            jnp.zeros((_H - _C - 1,), jnp.float32)])
    vpack = jnp.stack([b1, b2, vrow])
    fixed = lambda i: (0, 0)
    ot = pl.pallas_call(
        _mask_kernel,
        grid=(pl.cdiv(_N, _TILE),),
        in_specs=[
            pl.BlockSpec((_TILE, _D), lambda i: (i, 0)),
            pl.BlockSpec((_D, 2 * _H + _C), fixed),
            pl.BlockSpec((3, _H), fixed),
            pl.BlockSpec((_C, _TILE), lambda i: (0, i)),
        ],
        out_specs=pl.BlockSpec((_C, _TILE), lambda i: (0, i)),
        out_shape=jax.ShapeDtypeStruct((_C, _N), jnp.float32),
    )(x, wpack, vpack, gut)
    return ot.T


# speedup vs baseline: 1.6094x; 1.6094x over previous
import jax
import jax.numpy as jnp
from jax.experimental import pallas as pl
from jax.experimental.pallas import tpu as pltpu

_N, _D, _H, _C = 10000, 128, 128, 2
_TILE = 5120


def _mask_kernel(x_ref, w1_ref, b1_ref, a_ref, w2_ref, b2_ref, wc_ref,
                 bc_ref, gut_ref, o_ref):
    xt = x_ref[...].T
    h = jnp.dot(w1_ref[...].T, xt, preferred_element_type=jnp.float32)
    h = h + b1_ref[...].T
    a = a_ref[0, 0]
    h = jnp.where(h >= 0.0, h, a * h)
    h = jnp.dot(w2_ref[...].T, h, preferred_element_type=jnp.float32)
    h = h + b2_ref[...].T
    s = jnp.dot(wc_ref[...].T, h, preferred_element_type=jnp.float32)
    s = s + bc_ref[...].T
    g = -jnp.log(-jnp.log(gut_ref[...]))
    s = s + g
    m = jnp.max(s, axis=0, keepdims=True)
    e = jnp.exp(s - m)
    y = e / jnp.sum(e, axis=0, keepdims=True)
    hard1 = (y[1:2, :] > y[0:1, :]).astype(jnp.float32)
    o_ref[...] = jnp.concatenate([1.0 - hard1, hard1], axis=0)


def kernel(x, edge_index, W1, b1, prelu_a, W2, b2, Wc, bc, gumbel_u):
    del edge_index
    gut = gumbel_u.T
    b1r = b1.reshape(1, _H)
    b2r = b2.reshape(1, _H)
    bcr = bc.reshape(1, _C)
    ar = prelu_a.reshape(1, 1)
    fixed = lambda i: (0, 0)
    ot = pl.pallas_call(
        _mask_kernel,
        grid=(pl.cdiv(_N, _TILE),),
        in_specs=[
            pl.BlockSpec((_TILE, _D), lambda i: (i, 0)),
            pl.BlockSpec((_D, _H), fixed),
            pl.BlockSpec((1, _H), fixed),
            pl.BlockSpec((1, 1), fixed),
            pl.BlockSpec((_H, _H), fixed),
            pl.BlockSpec((1, _H), fixed),
            pl.BlockSpec((_H, _C), fixed),
            pl.BlockSpec((1, _C), fixed),
            pl.BlockSpec((_C, _TILE), lambda i: (0, i)),
        ],
        out_specs=pl.BlockSpec((_C, _TILE), lambda i: (0, i)),
        out_shape=jax.ShapeDtypeStruct((_C, _N), jnp.float32),
    )(x, W1, b1r, ar, W2, b2r, Wc, bcr, gut)
    return ot.T
